# Initial kernel scaffold; baseline (speedup 1.0000x reference)
import jax
import jax.numpy as jnp
from jax.experimental import pallas as pl


def kernel(inputs, table):
    raise NotImplementedError("write your pallas kernel here")


# speedup vs baseline: 1.8298x; 1.8298x over previous
import functools

import jax
import jax.numpy as jnp
from jax.experimental import pallas as pl
from jax.experimental.pallas import tpu as pltpu

MAX_REL = 128


def _rel_emb_kernel(table_ref, out_ref, summed_ref, *, S, D, BB):
    T = table_ref.shape[0]

    @pl.when(pl.program_id(0) == 0)
    def _compute_summed():
        i = jax.lax.broadcasted_iota(jnp.int32, (S, T), 0)
        t = jax.lax.broadcasted_iota(jnp.int32, (S, T), 1)
        t_lo = jnp.maximum(i - (S - 1 - MAX_REL), 0)
        t_hi = jnp.minimum(i + MAX_REL, 2 * MAX_REL)
        band = ((t >= t_lo) & (t <= t_hi)).astype(jnp.float32)
        lo_extra = jnp.where(t == 0, jnp.maximum((S - 1 - MAX_REL) - i, 0), 0)
        hi_extra = jnp.where(t == 2 * MAX_REL, jnp.maximum(i - MAX_REL, 0), 0)
        counts = band + lo_extra.astype(jnp.float32) + hi_extra.astype(jnp.float32)
        summed_ref[...] = jnp.dot(
            counts, table_ref[...], preferred_element_type=jnp.float32
        )

    out_ref[...] = jnp.broadcast_to(summed_ref[...][None, :, :], (BB, S, D))


def kernel(inputs, table):
    B, S = inputs.shape
    T, D = table.shape
    BB = 8
    grid = (B // BB,)
    out = pl.pallas_call(
        functools.partial(_rel_emb_kernel, S=S, D=D, BB=BB),
        grid=grid,
        in_specs=[pl.BlockSpec((T, D), lambda b: (0, 0))],
        out_specs=pl.BlockSpec((BB, S, D), lambda b: (b, 0, 0)),
        out_shape=jax.ShapeDtypeStruct((B, S, D), jnp.float32),
        scratch_shapes=[pltpu.VMEM((S, D), jnp.float32)],
    )(table)
    return out


# speedup vs baseline: 3.2559x; 1.7793x over previous
import functools

import jax
import jax.numpy as jnp
from jax.experimental import pallas as pl
from jax.experimental.pallas import tpu as pltpu

MAX_REL = 128


def _rel_emb_kernel(table_ref, out_ref, buf, sems, *, S, D, BB, B, NSEM):
    T = table_ref.shape[0]
    i = jax.lax.broadcasted_iota(jnp.int32, (S, T), 0)
    t = jax.lax.broadcasted_iota(jnp.int32, (S, T), 1)
    t_lo = jnp.maximum(i - (S - 1 - MAX_REL), 0)
    t_hi = jnp.minimum(i + MAX_REL, 2 * MAX_REL)
    band = ((t >= t_lo) & (t <= t_hi)).astype(jnp.float32)
    lo_extra = jnp.where(t == 0, jnp.maximum((S - 1 - MAX_REL) - i, 0), 0)
    hi_extra = jnp.where(t == 2 * MAX_REL, jnp.maximum(i - MAX_REL, 0), 0)
    counts = band + lo_extra.astype(jnp.float32) + hi_extra.astype(jnp.float32)
    summed = jnp.dot(counts, table_ref[...], preferred_element_type=jnp.float32)
    buf[...] = jnp.broadcast_to(summed[None, :, :], (BB, S, D))

    nblk = B // BB
    for k in range(nblk):
        if k >= NSEM:
            pltpu.make_async_copy(
                buf,
                out_ref.at[pl.ds((k - NSEM) * BB, BB)],
                sems.at[(k - NSEM) % NSEM],
            ).wait()
        pltpu.make_async_copy(
            buf, out_ref.at[pl.ds(k * BB, BB)], sems.at[k % NSEM]
        ).start()
    for k in range(max(nblk - NSEM, 0), nblk):
        pltpu.make_async_copy(
            buf, out_ref.at[pl.ds(k * BB, BB)], sems.at[k % NSEM]
        ).wait()


def kernel(inputs, table):
    B, S = inputs.shape
    T, D = table.shape
    BB = 32
    NSEM = 8
    out = pl.pallas_call(
        functools.partial(_rel_emb_kernel, S=S, D=D, BB=BB, B=B, NSEM=NSEM),
        in_specs=[pl.BlockSpec(memory_space=pltpu.MemorySpace.VMEM)],
        out_specs=pl.BlockSpec(memory_space=pl.ANY),
        out_shape=jax.ShapeDtypeStruct((B, S, D), jnp.float32),
        scratch_shapes=[
            pltpu.VMEM((BB, S, D), jnp.float32),
            pltpu.SemaphoreType.DMA((NSEM,)),
        ],
    )(table)
    return out


# speedup vs baseline: 3.3409x; 1.0261x over previous
import functools

import jax
import jax.numpy as jnp
from jax.experimental import pallas as pl
from jax.experimental.pallas import tpu as pltpu

MAX_REL = 128


def _rel_emb_kernel(table_ref, out_ref, buf, sems, *, S, D, BB, B, NSEM):
    T = table_ref.shape[0]
    i = jax.lax.broadcasted_iota(jnp.int32, (S, T), 0)
    t = jax.lax.broadcasted_iota(jnp.int32, (S, T), 1)
    t_lo = jnp.maximum(i - (S - 1 - MAX_REL), 0)
    t_hi = jnp.minimum(i + MAX_REL, 2 * MAX_REL)
    band = ((t >= t_lo) & (t <= t_hi)).astype(jnp.float32)
    lo_extra = jnp.where(t == 0, jnp.maximum((S - 1 - MAX_REL) - i, 0), 0)
    hi_extra = jnp.where(t == 2 * MAX_REL, jnp.maximum(i - MAX_REL, 0), 0)
    counts = band + lo_extra.astype(jnp.float32) + hi_extra.astype(jnp.float32)
    summed = jnp.dot(counts, table_ref[...], preferred_element_type=jnp.float32)
    buf[...] = jnp.broadcast_to(summed[None, :, :], (BB, S, D))

    nblk = B // BB
    for k in range(nblk):
        if k >= NSEM:
            pltpu.make_async_copy(
                buf,
                out_ref.at[pl.ds((k - NSEM) * BB, BB)],
                sems.at[(k - NSEM) % NSEM],
            ).wait()
        pltpu.make_async_copy(
            buf, out_ref.at[pl.ds(k * BB, BB)], sems.at[k % NSEM]
        ).start()
    for k in range(max(nblk - NSEM, 0), nblk):
        pltpu.make_async_copy(
            buf, out_ref.at[pl.ds(k * BB, BB)], sems.at[k % NSEM]
        ).wait()


def kernel(inputs, table):
    B, S = inputs.shape
    T, D = table.shape
    BB = 16
    NSEM = 16
    out = pl.pallas_call(
        functools.partial(_rel_emb_kernel, S=S, D=D, BB=BB, B=B, NSEM=NSEM),
        in_specs=[pl.BlockSpec(memory_space=pltpu.MemorySpace.VMEM)],
        out_specs=pl.BlockSpec(memory_space=pl.ANY),
        out_shape=jax.ShapeDtypeStruct((B, S, D), jnp.float32),
        scratch_shapes=[
            pltpu.VMEM((BB, S, D), jnp.float32),
            pltpu.SemaphoreType.DMA((NSEM,)),
        ],
    )(table)
    return out


# speedup vs baseline: 3.3519x; 1.0033x over previous
import functools

import jax
import jax.numpy as jnp
from jax.experimental import pallas as pl
from jax.experimental.pallas import tpu as pltpu

MAX_REL = 128


def _rel_emb_kernel(table_ref, out_ref, buf, sems, *, S, D, BB, B, NSEM):
    T = table_ref.shape[0]
    i = jax.lax.broadcasted_iota(jnp.int32, (S, T), 0)
    t = jax.lax.broadcasted_iota(jnp.int32, (S, T), 1)
    t_lo = jnp.maximum(i - (S - 1 - MAX_REL), 0)
    t_hi = jnp.minimum(i + MAX_REL, 2 * MAX_REL)
    band = ((t >= t_lo) & (t <= t_hi)).astype(jnp.float32)
    lo_extra = jnp.where(t == 0, jnp.maximum((S - 1 - MAX_REL) - i, 0), 0)
    hi_extra = jnp.where(t == 2 * MAX_REL, jnp.maximum(i - MAX_REL, 0), 0)
    counts = band + lo_extra.astype(jnp.float32) + hi_extra.astype(jnp.float32)
    summed = jnp.dot(counts, table_ref[...], preferred_element_type=jnp.float32)
    buf[...] = jnp.broadcast_to(summed[None, :, :], (BB, S, D))

    nblk = B // BB
    for k in range(nblk):
        if k >= NSEM:
            pltpu.make_async_copy(
                buf,
                out_ref.at[pl.ds((k - NSEM) * BB, BB)],
                sems.at[(k - NSEM) % NSEM],
            ).wait()
        pltpu.make_async_copy(
            buf, out_ref.at[pl.ds(k * BB, BB)], sems.at[k % NSEM]
        ).start()
    for k in range(max(nblk - NSEM, 0), nblk):
        pltpu.make_async_copy(
            buf, out_ref.at[pl.ds(k * BB, BB)], sems.at[k % NSEM]
        ).wait()


def kernel(inputs, table):
    B, S = inputs.shape
    T, D = table.shape
    BB = 8
    NSEM = 32
    out = pl.pallas_call(
        functools.partial(_rel_emb_kernel, S=S, D=D, BB=BB, B=B, NSEM=NSEM),
        in_specs=[pl.BlockSpec(memory_space=pltpu.MemorySpace.VMEM)],
        out_specs=pl.BlockSpec(memory_space=pl.ANY),
        out_shape=jax.ShapeDtypeStruct((B, S, D), jnp.float32),
        scratch_shapes=[
            pltpu.VMEM((BB, S, D), jnp.float32),
            pltpu.SemaphoreType.DMA((NSEM,)),
        ],
    )(table)
    return out
